# initial kernel scaffold (unmeasured)
import numpy as np
import jax
import jax.numpy as jnp
from jax import lax
from jax.experimental import pallas as pl
from jax.experimental.pallas import tpu as pltpu

N_DEV = 4
B, SQ, D = 2, 512, 1024
HQ_LOC, DH = 8, 128
ROWS = B * SQ
CHUNK = ROWS // N_DEV
SCALE = 0.08838834764831843


def _rope_tables():
    inv = 1.0 / (10000.0 ** (np.arange(0, DH, 2) / DH))
    pos = np.arange(SQ)[:, None] * inv[None, :]
    cos = np.repeat(np.cos(pos), 2, axis=-1)
    sin = np.repeat(np.sin(pos), 2, axis=-1)
    cos_t = np.tile(cos, (B, HQ_LOC))
    sin_t = np.tile(sin, (B, HQ_LOC))
    even = (np.arange(D) % 2 == 0)[None, :]
    sin_a = np.where(even, -sin_t, 0.0)
    sin_b = np.where(even, 0.0, sin_t)
    return (
        cos_t.astype(np.float32),
        sin_a.astype(np.float32),
        sin_b.astype(np.float32),
    )


_COS_T, _SIN_A, _SIN_B = _rope_tables()


def _body(
    xf_ref, wq_ref, wk_ref, wv_ref, wo_ref, cos_ref, sa_ref, sb_ref,
    out_ref,
    q_scr, k_scr, v_scr, ctx_scr, part_scr, rs_buf,
    rs_send, rs_recv, ag_send, ag_recv,
):
    my = lax.axis_index("i")
    left = (my - 1) % N_DEV
    right = (my + 1) % N_DEV

    barrier_sem = pltpu.get_barrier_semaphore()
    for nbr in (left, right):
        pl.semaphore_signal(
            barrier_sem, inc=1,
            device_id=(nbr,), device_id_type=pl.DeviceIdType.MESH,
        )
    pl.semaphore_wait(barrier_sem, 2)

    xf = xf_ref[...]

    def rope(t):
        return (
            t * cos_ref[...]
            + pltpu.roll(t, D - 1, axis=1) * sa_ref[...]
            + pltpu.roll(t, 1, axis=1) * sb_ref[...]
        )

    q_scr[...] = rope(jnp.dot(xf, wq_ref[...], preferred_element_type=jnp.float32))
    k_scr[...] = rope(jnp.dot(xf, wk_ref[...], preferred_element_type=jnp.float32))
    v_scr[...] = jnp.dot(xf, wv_ref[...], preferred_element_type=jnp.float32)

    for b in range(B):
        for h in range(HQ_LOC):
            r0, c0 = b * SQ, h * DH
            qh = q_scr[r0:r0 + SQ, c0:c0 + DH]
            kh = k_scr[r0:r0 + SQ, c0:c0 + DH]
            vh = v_scr[r0:r0 + SQ, c0:c0 + DH]
            s = lax.dot_general(
                qh, kh, (((1,), (1,)), ((), ())),
                preferred_element_type=jnp.float32,
            ) * SCALE
            m = jnp.max(s, axis=1, keepdims=True)
            e = jnp.exp(s - m)
            w = e / jnp.sum(e, axis=1, keepdims=True)
            ctx_scr[r0:r0 + SQ, c0:c0 + DH] = jnp.dot(
                w, vh, preferred_element_type=jnp.float32
            )

    part_scr[...] = jnp.dot(
        ctx_scr[...], wo_ref[...], preferred_element_type=jnp.float32
    )

    for s in range(N_DEV - 1):
        send_c = (my - s) % N_DEV
        recv_c = (my - s - 1) % N_DEV
        rdma = pltpu.make_async_remote_copy(
            src_ref=part_scr.at[pl.ds(send_c * CHUNK, CHUNK), :],
            dst_ref=rs_buf.at[s],
            send_sem=rs_send.at[s],
            recv_sem=rs_recv.at[s],
            device_id=(right,),
            device_id_type=pl.DeviceIdType.MESH,
        )
        rdma.start()
        rdma.wait()
        part_scr[pl.ds(recv_c * CHUNK, CHUNK), :] = (
            part_scr[pl.ds(recv_c * CHUNK, CHUNK), :] + rs_buf[s]
        )

    own = (my + 1) % N_DEV
    out_ref[pl.ds(own * CHUNK, CHUNK), :] = part_scr[pl.ds(own * CHUNK, CHUNK), :]
    for t in range(N_DEV - 1):
        g = (my + 1 - t) % N_DEV
        rdma = pltpu.make_async_remote_copy(
            src_ref=out_ref.at[pl.ds(g * CHUNK, CHUNK), :],
            dst_ref=out_ref.at[pl.ds(g * CHUNK, CHUNK), :],
            send_sem=ag_send.at[t],
            recv_sem=ag_recv.at[t],
            device_id=(right,),
            device_id_type=pl.DeviceIdType.MESH,
        )
        rdma.start()
        rdma.wait()


def kernel(x, Wq, Wk, Wv, Wo):
    xf = jnp.reshape(x, (ROWS, D))
    out = pl.pallas_call(
        _body,
        out_shape=jax.ShapeDtypeStruct((ROWS, D), jnp.float32),
        in_specs=[pl.BlockSpec(memory_space=pltpu.VMEM)] * 8,
        out_specs=pl.BlockSpec(memory_space=pltpu.VMEM),
        scratch_shapes=[
            pltpu.VMEM((ROWS, D), jnp.float32),
            pltpu.VMEM((ROWS, D), jnp.float32),
            pltpu.VMEM((ROWS, D), jnp.float32),
            pltpu.VMEM((ROWS, D), jnp.float32),
            pltpu.VMEM((ROWS, D), jnp.float32),
            pltpu.VMEM((N_DEV - 1, CHUNK, D), jnp.float32),
            pltpu.SemaphoreType.DMA((N_DEV - 1,)),
            pltpu.SemaphoreType.DMA((N_DEV - 1,)),
            pltpu.SemaphoreType.DMA((N_DEV - 1,)),
            pltpu.SemaphoreType.DMA((N_DEV - 1,)),
        ],
        compiler_params=pltpu.CompilerParams(collective_id=0),
    )(
        xf, Wq, Wk, Wv, Wo,
        jnp.asarray(_COS_T), jnp.asarray(_SIN_A), jnp.asarray(_SIN_B),
    )
    return jnp.reshape(out, (B, SQ, D))


# baseline (device time: 124065 ns/iter reference)
import numpy as np
import jax
import jax.numpy as jnp
from jax import lax
from jax.experimental import pallas as pl
from jax.experimental.pallas import tpu as pltpu

N_DEV = 4
B, SQ, D = 2, 512, 1024
HQ_LOC, DH = 8, 128
ROWS = B * SQ
CHUNK = ROWS // N_DEV
SCALE = 0.08838834764831843


def _rope_tables():
    inv = 1.0 / (10000.0 ** (np.arange(0, DH, 2) / DH))
    pos = np.arange(SQ)[:, None] * inv[None, :]
    cos = np.repeat(np.cos(pos), 2, axis=-1)
    sin = np.repeat(np.sin(pos), 2, axis=-1)
    cos_t = np.tile(cos, (B, HQ_LOC))
    sin_t = np.tile(sin, (B, HQ_LOC))
    even = (np.arange(D) % 2 == 0)[None, :]
    sin_a = np.where(even, -sin_t, 0.0)
    sin_b = np.where(even, 0.0, sin_t)
    return (
        cos_t.astype(np.float32),
        sin_a.astype(np.float32),
        sin_b.astype(np.float32),
    )


_COS_T, _SIN_A, _SIN_B = _rope_tables()


def _body(
    xf_ref, wq_ref, wk_ref, wv_ref, wo_ref, cos_ref, sa_ref, sb_ref,
    out_ref,
    q_scr, k_scr, v_scr, ctx_scr, part_scr, rs_buf,
    rs_send, rs_recv, ag_send, ag_recv,
):
    my = lax.axis_index("i")
    left = (my - 1) % N_DEV
    right = (my + 1) % N_DEV

    barrier_sem = pltpu.get_barrier_semaphore()
    for nbr in (left, right):
        pl.semaphore_signal(
            barrier_sem, inc=1,
            device_id=(nbr,), device_id_type=pl.DeviceIdType.MESH,
        )
    pl.semaphore_wait(barrier_sem, 2)

    xf = xf_ref[...]

    def rope(t):
        return (
            t * cos_ref[...]
            + pltpu.roll(t, D - 1, axis=1) * sa_ref[...]
            + pltpu.roll(t, 1, axis=1) * sb_ref[...]
        )

    q_scr[...] = rope(jnp.dot(xf, wq_ref[...], preferred_element_type=jnp.float32))
    k_scr[...] = rope(jnp.dot(xf, wk_ref[...], preferred_element_type=jnp.float32))
    v_scr[...] = jnp.dot(xf, wv_ref[...], preferred_element_type=jnp.float32)

    for b in range(B):
        for h in range(HQ_LOC):
            r0, c0 = b * SQ, h * DH
            qh = q_scr[r0:r0 + SQ, c0:c0 + DH]
            kh = k_scr[r0:r0 + SQ, c0:c0 + DH]
            vh = v_scr[r0:r0 + SQ, c0:c0 + DH]
            s = lax.dot_general(
                qh, kh, (((1,), (1,)), ((), ())),
                preferred_element_type=jnp.float32,
            ) * SCALE
            m = jnp.max(s, axis=1, keepdims=True)
            e = jnp.exp(s - m)
            w = e / jnp.sum(e, axis=1, keepdims=True)
            ctx_scr[r0:r0 + SQ, c0:c0 + DH] = jnp.dot(
                w, vh, preferred_element_type=jnp.float32
            )

    part_scr[...] = jnp.dot(
        ctx_scr[...], wo_ref[...], preferred_element_type=jnp.float32
    )

    for s in range(N_DEV - 1):
        send_c = (my - s) % N_DEV
        recv_c = (my - s - 1) % N_DEV
        rdma = pltpu.make_async_remote_copy(
            src_ref=part_scr.at[pl.ds(send_c * CHUNK, CHUNK), :],
            dst_ref=rs_buf.at[s],
            send_sem=rs_send.at[s],
            recv_sem=rs_recv.at[s],
            device_id=(right,),
            device_id_type=pl.DeviceIdType.MESH,
        )
        rdma.start()
        rdma.wait()
        part_scr[pl.ds(recv_c * CHUNK, CHUNK), :] = (
            part_scr[pl.ds(recv_c * CHUNK, CHUNK), :] + rs_buf[s]
        )

    own = (my + 1) % N_DEV
    out_ref[pl.ds(own * CHUNK, CHUNK), :] = part_scr[pl.ds(own * CHUNK, CHUNK), :]
    for t in range(N_DEV - 1):
        g = (my + 1 - t) % N_DEV
        rdma = pltpu.make_async_remote_copy(
            src_ref=out_ref.at[pl.ds(g * CHUNK, CHUNK), :],
            dst_ref=out_ref.at[pl.ds(g * CHUNK, CHUNK), :],
            send_sem=ag_send.at[t],
            recv_sem=ag_recv.at[t],
            device_id=(right,),
            device_id_type=pl.DeviceIdType.MESH,
        )
        rdma.start()
        rdma.wait()


def kernel(x, Wq, Wk, Wv, Wo):
    xf = jnp.reshape(x, (ROWS, D))
    out = pl.pallas_call(
        _body,
        out_shape=jax.ShapeDtypeStruct((ROWS, D), jnp.float32),
        in_specs=[pl.BlockSpec(memory_space=pltpu.VMEM)] * 8,
        out_specs=pl.BlockSpec(memory_space=pltpu.VMEM),
        scratch_shapes=[
            pltpu.VMEM((ROWS, D), jnp.float32),
            pltpu.VMEM((ROWS, D), jnp.float32),
            pltpu.VMEM((ROWS, D), jnp.float32),
            pltpu.VMEM((ROWS, D), jnp.float32),
            pltpu.VMEM((ROWS, D), jnp.float32),
            pltpu.VMEM((N_DEV - 1, CHUNK, D), jnp.float32),
            pltpu.SemaphoreType.DMA((N_DEV - 1,)),
            pltpu.SemaphoreType.DMA((N_DEV - 1,)),
            pltpu.SemaphoreType.DMA((N_DEV - 1,)),
            pltpu.SemaphoreType.DMA((N_DEV - 1,)),
        ],
        compiler_params=pltpu.CompilerParams(
            collective_id=0,
            vmem_limit_bytes=100 * 1024 * 1024,
        ),
    )(
        xf, Wq, Wk, Wv, Wo,
        jnp.asarray(_COS_T), jnp.asarray(_SIN_A), jnp.asarray(_SIN_B),
    )
    return jnp.reshape(out, (B, SQ, D))


# device time: 86744 ns/iter; 1.4302x vs baseline; 1.4302x over previous
import numpy as np
import jax
import jax.numpy as jnp
from jax import lax
from jax.experimental import pallas as pl
from jax.experimental.pallas import tpu as pltpu

N_DEV = 4
B, SQ, D = 2, 512, 1024
HQ_LOC, DH = 8, 128
ROWS = B * SQ
CHUNK = ROWS // N_DEV
SCALE = 0.08838834764831843


def _rope_tables():
    inv = 1.0 / (10000.0 ** (np.arange(0, DH, 2) / DH))
    pos = np.arange(SQ)[:, None] * inv[None, :]
    cos = np.repeat(np.cos(pos), 2, axis=-1)
    sin = np.repeat(np.sin(pos), 2, axis=-1)
    cos_t = np.tile(cos, (B, HQ_LOC))
    sin_t = np.tile(sin, (B, HQ_LOC))
    even = (np.arange(D) % 2 == 0)[None, :]
    sin_a = np.where(even, -sin_t, 0.0)
    sin_b = np.where(even, 0.0, sin_t)
    return (
        cos_t.astype(np.float32),
        sin_a.astype(np.float32),
        sin_b.astype(np.float32),
    )


_COS_T, _SIN_A, _SIN_B = _rope_tables()


def _body(
    xf_ref, wq_ref, wk_ref, wv_ref, wo_ref, cos_ref, sa_ref, sb_ref,
    out_ref,
    k_scr, v_scr, part_scr, gather_scr, rs_buf,
    rs_send, rs_recv, ag_send, ag_recv,
):
    my = lax.axis_index("i")

    barrier_sem = pltpu.get_barrier_semaphore()
    for d in range(1, N_DEV):
        pl.semaphore_signal(
            barrier_sem, inc=1,
            device_id=((my + d) % N_DEV,),
            device_id_type=pl.DeviceIdType.MESH,
        )
    pl.semaphore_wait(barrier_sem, N_DEV - 1)

    def rope(t, row_ds):
        return (
            t * cos_ref[row_ds, :]
            + pltpu.roll(t, D - 1, axis=1) * sa_ref[row_ds, :]
            + pltpu.roll(t, 1, axis=1) * sb_ref[row_ds, :]
        )

    all_rows = pl.ds(0, ROWS)
    k_scr[...] = rope(
        jnp.dot(xf_ref[...], wk_ref[...], preferred_element_type=jnp.float32),
        all_rows,
    )
    v_scr[...] = jnp.dot(
        xf_ref[...], wv_ref[...], preferred_element_type=jnp.float32
    )

    def compute_chunk(c):
        rows = pl.ds(c * CHUNK, CHUNK)
        b = c // (SQ // CHUNK)
        brows = pl.ds(b * SQ, SQ)
        q = rope(
            jnp.dot(
                xf_ref[rows, :], wq_ref[...],
                preferred_element_type=jnp.float32,
            ),
            rows,
        )
        acc = jnp.zeros((CHUNK, D), jnp.float32)
        for h in range(HQ_LOC):
            c0 = h * DH
            qh = q[:, c0:c0 + DH]
            kh = k_scr[brows, c0:c0 + DH]
            vh = v_scr[brows, c0:c0 + DH]
            s = lax.dot_general(
                qh, kh, (((1,), (1,)), ((), ())),
                preferred_element_type=jnp.float32,
            ) * SCALE
            m = jnp.max(s, axis=1, keepdims=True)
            e = jnp.exp(s - m)
            w = e / jnp.sum(e, axis=1, keepdims=True)
            ctxh = jnp.dot(w, vh, preferred_element_type=jnp.float32)
            acc = acc + jnp.dot(
                ctxh, wo_ref[c0:c0 + DH, :],
                preferred_element_type=jnp.float32,
            )
        return acc

    rs_rdmas = []
    for d in (1, 2, 3):
        c = (my + d) % N_DEV
        rows = pl.ds(c * CHUNK, CHUNK)
        part_scr[rows, :] = compute_chunk(c)
        rdma = pltpu.make_async_remote_copy(
            src_ref=part_scr.at[rows, :],
            dst_ref=rs_buf.at[3 - d],
            send_sem=rs_send.at[d - 1],
            recv_sem=rs_recv.at[3 - d],
            device_id=(c,),
            device_id_type=pl.DeviceIdType.MESH,
        )
        rdma.start()
        rs_rdmas.append(rdma)

    own_rows = pl.ds(my * CHUNK, CHUNK)
    own = compute_chunk(my)
    for slot in range(N_DEV - 1):
        recv = pltpu.make_async_remote_copy(
            src_ref=part_scr.at[pl.ds(0, CHUNK), :],
            dst_ref=rs_buf.at[slot],
            send_sem=rs_send.at[slot],
            recv_sem=rs_recv.at[slot],
            device_id=(my,),
            device_id_type=pl.DeviceIdType.MESH,
        )
        recv.wait_recv()
    gather_scr[own_rows, :] = own + rs_buf[0] + rs_buf[1] + rs_buf[2]

    ag_rdmas = []
    for d in (1, 2, 3):
        peer = (my + d) % N_DEV
        rdma = pltpu.make_async_remote_copy(
            src_ref=gather_scr.at[own_rows, :],
            dst_ref=gather_scr.at[own_rows, :],
            send_sem=ag_send.at[d - 1],
            recv_sem=ag_recv.at[3 - d],
            device_id=(peer,),
            device_id_type=pl.DeviceIdType.MESH,
        )
        rdma.start()
        ag_rdmas.append(rdma)
    for slot in range(N_DEV - 1):
        recv = pltpu.make_async_remote_copy(
            src_ref=gather_scr.at[pl.ds(0, CHUNK), :],
            dst_ref=gather_scr.at[own_rows, :],
            send_sem=ag_send.at[slot],
            recv_sem=ag_recv.at[slot],
            device_id=(my,),
            device_id_type=pl.DeviceIdType.MESH,
        )
        recv.wait_recv()

    for rdma in rs_rdmas + ag_rdmas:
        rdma.wait_send()

    out_ref[...] = gather_scr[...]


def kernel(x, Wq, Wk, Wv, Wo):
    xf = jnp.reshape(x, (ROWS, D))
    out = pl.pallas_call(
        _body,
        out_shape=jax.ShapeDtypeStruct((ROWS, D), jnp.float32),
        in_specs=[pl.BlockSpec(memory_space=pltpu.VMEM)] * 8,
        out_specs=pl.BlockSpec(memory_space=pltpu.VMEM),
        scratch_shapes=[
            pltpu.VMEM((ROWS, D), jnp.float32),
            pltpu.VMEM((ROWS, D), jnp.float32),
            pltpu.VMEM((ROWS, D), jnp.float32),
            pltpu.VMEM((ROWS, D), jnp.float32),
            pltpu.VMEM((N_DEV - 1, CHUNK, D), jnp.float32),
            pltpu.SemaphoreType.DMA((N_DEV - 1,)),
            pltpu.SemaphoreType.DMA((N_DEV - 1,)),
            pltpu.SemaphoreType.DMA((N_DEV - 1,)),
            pltpu.SemaphoreType.DMA((N_DEV - 1,)),
        ],
        compiler_params=pltpu.CompilerParams(
            collective_id=0,
            vmem_limit_bytes=100 * 1024 * 1024,
        ),
    )(
        xf, Wq, Wk, Wv, Wo,
        jnp.asarray(_COS_T), jnp.asarray(_SIN_A), jnp.asarray(_SIN_B),
    )
    return jnp.reshape(out, (B, SQ, D))


# device time: 60161 ns/iter; 2.0622x vs baseline; 1.4419x over previous
import numpy as np
import jax
import jax.numpy as jnp
from jax import lax
from jax.experimental import pallas as pl
from jax.experimental.pallas import tpu as pltpu

N_DEV = 4
B, SQ, D = 2, 512, 1024
HQ_LOC, DH = 8, 128
ROWS = B * SQ
CHUNK = ROWS // N_DEV
SCALE = 0.08838834764831843


def _rope_tables():
    inv = 1.0 / (10000.0 ** (np.arange(0, DH, 2) / DH))
    pos = np.arange(SQ)[:, None] * inv[None, :]
    cos = np.repeat(np.cos(pos), 2, axis=-1)
    sin = np.repeat(np.sin(pos), 2, axis=-1)
    cos_t = np.tile(cos, (B, HQ_LOC))
    sin_t = np.tile(sin, (B, HQ_LOC))
    even = (np.arange(D) % 2 == 0)[None, :]
    sin_a = np.where(even, -sin_t, 0.0)
    sin_b = np.where(even, 0.0, sin_t)
    return (
        cos_t.astype(np.float32),
        sin_a.astype(np.float32),
        sin_b.astype(np.float32),
    )


_COS_T, _SIN_A, _SIN_B = _rope_tables()


def _body(
    xf_ref, wq_ref, wk_ref, wv_ref, wo_ref, cos_ref, sa_ref, sb_ref,
    out_ref,
    k_scr, v_scr, part16_scr, ag16_scr, rs_buf,
    rs_send, rs_recv, ag_send, ag_recv,
):
    my = lax.axis_index("i")

    barrier_sem = pltpu.get_barrier_semaphore()
    for d in range(1, N_DEV):
        pl.semaphore_signal(
            barrier_sem, inc=1,
            device_id=((my + d) % N_DEV,),
            device_id_type=pl.DeviceIdType.MESH,
        )
    pl.semaphore_wait(barrier_sem, N_DEV - 1)

    def rope(t, row_ds):
        return (
            t * cos_ref[row_ds, :]
            + pltpu.roll(t, D - 1, axis=1) * sa_ref[row_ds, :]
            + pltpu.roll(t, 1, axis=1) * sb_ref[row_ds, :]
        )

    all_rows = pl.ds(0, ROWS)
    k_scr[...] = rope(
        jnp.dot(xf_ref[...], wk_ref[...], preferred_element_type=jnp.float32),
        all_rows,
    )
    v_scr[...] = jnp.dot(
        xf_ref[...], wv_ref[...], preferred_element_type=jnp.float32
    )

    def compute_chunk(c):
        rows = pl.ds(c * CHUNK, CHUNK)
        b = c // (SQ // CHUNK)
        brows = pl.ds(b * SQ, SQ)
        q = rope(
            jnp.dot(
                xf_ref[rows, :], wq_ref[...],
                preferred_element_type=jnp.float32,
            ),
            rows,
        )
        acc = jnp.zeros((CHUNK, D), jnp.float32)
        for h in range(HQ_LOC):
            c0 = h * DH
            qh = q[:, c0:c0 + DH]
            kh = k_scr[brows, c0:c0 + DH]
            vh = v_scr[brows, c0:c0 + DH]
            s = lax.dot_general(
                qh, kh, (((1,), (1,)), ((), ())),
                preferred_element_type=jnp.float32,
            ) * SCALE
            e = jnp.exp(s)
            denom = jnp.sum(e, axis=1, keepdims=True)
            ctxh = jnp.dot(e, vh, preferred_element_type=jnp.float32) / denom
            acc = acc + jnp.dot(
                ctxh, wo_ref[c0:c0 + DH, :],
                preferred_element_type=jnp.float32,
            )
        return acc

    rs_rdmas = []
    for d in (1, 2, 3):
        c = (my + d) % N_DEV
        rows = pl.ds(c * CHUNK, CHUNK)
        part16_scr[rows, :] = compute_chunk(c).astype(jnp.bfloat16)
        rdma = pltpu.make_async_remote_copy(
            src_ref=part16_scr.at[rows, :],
            dst_ref=rs_buf.at[3 - d],
            send_sem=rs_send.at[d - 1],
            recv_sem=rs_recv.at[3 - d],
            device_id=(c,),
            device_id_type=pl.DeviceIdType.MESH,
        )
        rdma.start()
        rs_rdmas.append(rdma)

    own_rows = pl.ds(my * CHUNK, CHUNK)
    own = compute_chunk(my)
    for slot in range(N_DEV - 1):
        recv = pltpu.make_async_remote_copy(
            src_ref=part16_scr.at[pl.ds(0, CHUNK), :],
            dst_ref=rs_buf.at[slot],
            send_sem=rs_send.at[slot],
            recv_sem=rs_recv.at[slot],
            device_id=(my,),
            device_id_type=pl.DeviceIdType.MESH,
        )
        recv.wait_recv()
    reduced = (
        own
        + rs_buf[0].astype(jnp.float32)
        + rs_buf[1].astype(jnp.float32)
        + rs_buf[2].astype(jnp.float32)
    )
    ag16_scr[own_rows, :] = reduced.astype(jnp.bfloat16)

    ag_rdmas = []
    for d in (1, 2, 3):
        peer = (my + d) % N_DEV
        rdma = pltpu.make_async_remote_copy(
            src_ref=ag16_scr.at[own_rows, :],
            dst_ref=ag16_scr.at[own_rows, :],
            send_sem=ag_send.at[d - 1],
            recv_sem=ag_recv.at[3 - d],
            device_id=(peer,),
            device_id_type=pl.DeviceIdType.MESH,
        )
        rdma.start()
        ag_rdmas.append(rdma)
    for slot in range(N_DEV - 1):
        recv = pltpu.make_async_remote_copy(
            src_ref=ag16_scr.at[pl.ds(0, CHUNK), :],
            dst_ref=ag16_scr.at[own_rows, :],
            send_sem=ag_send.at[slot],
            recv_sem=ag_recv.at[slot],
            device_id=(my,),
            device_id_type=pl.DeviceIdType.MESH,
        )
        recv.wait_recv()

    for rdma in rs_rdmas + ag_rdmas:
        rdma.wait_send()

    out_ref[...] = ag16_scr[...].astype(jnp.float32)


def kernel(x, Wq, Wk, Wv, Wo):
    xf = jnp.reshape(x, (ROWS, D))
    out = pl.pallas_call(
        _body,
        out_shape=jax.ShapeDtypeStruct((ROWS, D), jnp.float32),
        in_specs=[pl.BlockSpec(memory_space=pltpu.VMEM)] * 8,
        out_specs=pl.BlockSpec(memory_space=pltpu.VMEM),
        scratch_shapes=[
            pltpu.VMEM((ROWS, D), jnp.float32),
            pltpu.VMEM((ROWS, D), jnp.float32),
            pltpu.VMEM((ROWS, D), jnp.bfloat16),
            pltpu.VMEM((ROWS, D), jnp.bfloat16),
            pltpu.VMEM((N_DEV - 1, CHUNK, D), jnp.bfloat16),
            pltpu.SemaphoreType.DMA((N_DEV - 1,)),
            pltpu.SemaphoreType.DMA((N_DEV - 1,)),
            pltpu.SemaphoreType.DMA((N_DEV - 1,)),
            pltpu.SemaphoreType.DMA((N_DEV - 1,)),
        ],
        compiler_params=pltpu.CompilerParams(
            collective_id=0,
            vmem_limit_bytes=100 * 1024 * 1024,
        ),
    )(
        xf, Wq, Wk, Wv, Wo,
        jnp.asarray(_COS_T), jnp.asarray(_SIN_A), jnp.asarray(_SIN_B),
    )
    return jnp.reshape(out, (B, SQ, D))


# device time: 59887 ns/iter; 2.0717x vs baseline; 1.0046x over previous
import numpy as np
import jax
import jax.numpy as jnp
from jax import lax
from jax.experimental import pallas as pl
from jax.experimental.pallas import tpu as pltpu

N_DEV = 4
B, SQ, D = 2, 512, 1024
HQ_LOC, DH = 8, 128
ROWS = B * SQ
CHUNK = ROWS // N_DEV
SCALE = 0.08838834764831843


def _rope_tables():
    inv = 1.0 / (10000.0 ** (np.arange(0, DH, 2) / DH))
    pos = np.arange(SQ)[:, None] * inv[None, :]
    cos = np.repeat(np.cos(pos), 2, axis=-1)
    sin = np.repeat(np.sin(pos), 2, axis=-1)
    cos_t = np.tile(cos, (B, HQ_LOC))
    sin_t = np.tile(sin, (B, HQ_LOC))
    even = (np.arange(D) % 2 == 0)[None, :]
    sin_a = np.where(even, -sin_t, 0.0)
    sin_b = np.where(even, 0.0, sin_t)
    return (
        cos_t.astype(np.float32),
        sin_a.astype(np.float32),
        sin_b.astype(np.float32),
    )


_COS_T, _SIN_A, _SIN_B = _rope_tables()


def _body(
    xf_ref, wq_ref, wk_ref, wv_ref, wo_ref, cos_ref, sa_ref, sb_ref,
    out_ref,
    k_scr, v_scr, part16_scr, ag16_scr, rs_buf,
    rs_send, rs_recv, ag_send, ag_recv,
):
    my = lax.axis_index("i")

    barrier_sem = pltpu.get_barrier_semaphore()
    for d in range(1, N_DEV):
        pl.semaphore_signal(
            barrier_sem, inc=1,
            device_id=((my + d) % N_DEV,),
            device_id_type=pl.DeviceIdType.MESH,
        )
    pl.semaphore_wait(barrier_sem, N_DEV - 1)

    def rope(t, row_ds):
        return (
            t * cos_ref[row_ds, :]
            + pltpu.roll(t, D - 1, axis=1) * sa_ref[row_ds, :]
            + pltpu.roll(t, 1, axis=1) * sb_ref[row_ds, :]
        )

    bf16 = jnp.bfloat16
    xf16 = xf_ref[...].astype(bf16)
    wq16 = wq_ref[...].astype(bf16)
    wo16 = wo_ref[...].astype(bf16)
    all_rows = pl.ds(0, ROWS)
    k_scr[...] = rope(
        jnp.dot(
            xf16, wk_ref[...].astype(bf16),
            preferred_element_type=jnp.float32,
        ),
        all_rows,
    ).astype(bf16)
    v_scr[...] = jnp.dot(
        xf16, wv_ref[...].astype(bf16), preferred_element_type=jnp.float32
    ).astype(bf16)

    def compute_chunk(c):
        rows = pl.ds(c * CHUNK, CHUNK)
        b = c // (SQ // CHUNK)
        brows = pl.ds(b * SQ, SQ)
        q = rope(
            jnp.dot(
                xf_ref[rows, :].astype(bf16),
                wq16,
                preferred_element_type=jnp.float32,
            ),
            rows,
        ).astype(bf16)
        acc = jnp.zeros((CHUNK, D), jnp.float32)
        for h in range(HQ_LOC):
            c0 = h * DH
            qh = q[:, c0:c0 + DH]
            kh = k_scr[brows, c0:c0 + DH]
            vh = v_scr[brows, c0:c0 + DH]
            s = lax.dot_general(
                qh, kh, (((1,), (1,)), ((), ())),
                preferred_element_type=jnp.float32,
            ) * SCALE
            e = jnp.exp(s)
            denom = jnp.sum(e, axis=1, keepdims=True)
            ctxh = (
                jnp.dot(
                    e.astype(bf16), vh, preferred_element_type=jnp.float32
                )
                / denom
            )
            acc = acc + jnp.dot(
                ctxh.astype(bf16), wo16[c0:c0 + DH, :],
                preferred_element_type=jnp.float32,
            )
        return acc

    rs_rdmas = []
    for d in (1, 2, 3):
        c = (my + d) % N_DEV
        rows = pl.ds(c * CHUNK, CHUNK)
        part16_scr[rows, :] = compute_chunk(c).astype(jnp.bfloat16)
        rdma = pltpu.make_async_remote_copy(
            src_ref=part16_scr.at[rows, :],
            dst_ref=rs_buf.at[3 - d],
            send_sem=rs_send.at[d - 1],
            recv_sem=rs_recv.at[3 - d],
            device_id=(c,),
            device_id_type=pl.DeviceIdType.MESH,
        )
        rdma.start()
        rs_rdmas.append(rdma)

    own_rows = pl.ds(my * CHUNK, CHUNK)
    own = compute_chunk(my)
    for slot in range(N_DEV - 1):
        recv = pltpu.make_async_remote_copy(
            src_ref=part16_scr.at[pl.ds(0, CHUNK), :],
            dst_ref=rs_buf.at[slot],
            send_sem=rs_send.at[slot],
            recv_sem=rs_recv.at[slot],
            device_id=(my,),
            device_id_type=pl.DeviceIdType.MESH,
        )
        recv.wait_recv()
    reduced = (
        own
        + rs_buf[0].astype(jnp.float32)
        + rs_buf[1].astype(jnp.float32)
        + rs_buf[2].astype(jnp.float32)
    )
    ag16_scr[own_rows, :] = reduced.astype(jnp.bfloat16)

    ag_rdmas = []
    for d in (1, 2, 3):
        peer = (my + d) % N_DEV
        rdma = pltpu.make_async_remote_copy(
            src_ref=ag16_scr.at[own_rows, :],
            dst_ref=ag16_scr.at[own_rows, :],
            send_sem=ag_send.at[d - 1],
            recv_sem=ag_recv.at[3 - d],
            device_id=(peer,),
            device_id_type=pl.DeviceIdType.MESH,
        )
        rdma.start()
        ag_rdmas.append(rdma)
    for slot in range(N_DEV - 1):
        recv = pltpu.make_async_remote_copy(
            src_ref=ag16_scr.at[pl.ds(0, CHUNK), :],
            dst_ref=ag16_scr.at[own_rows, :],
            send_sem=ag_send.at[slot],
            recv_sem=ag_recv.at[slot],
            device_id=(my,),
            device_id_type=pl.DeviceIdType.MESH,
        )
        recv.wait_recv()

    for rdma in rs_rdmas + ag_rdmas:
        rdma.wait_send()

    out_ref[...] = ag16_scr[...].astype(jnp.float32)


def kernel(x, Wq, Wk, Wv, Wo):
    xf = jnp.reshape(x, (ROWS, D))
    out = pl.pallas_call(
        _body,
        out_shape=jax.ShapeDtypeStruct((ROWS, D), jnp.float32),
        in_specs=[pl.BlockSpec(memory_space=pltpu.VMEM)] * 8,
        out_specs=pl.BlockSpec(memory_space=pltpu.VMEM),
        scratch_shapes=[
            pltpu.VMEM((ROWS, D), jnp.bfloat16),
            pltpu.VMEM((ROWS, D), jnp.bfloat16),
            pltpu.VMEM((ROWS, D), jnp.bfloat16),
            pltpu.VMEM((ROWS, D), jnp.bfloat16),
            pltpu.VMEM((N_DEV - 1, CHUNK, D), jnp.bfloat16),
            pltpu.SemaphoreType.DMA((N_DEV - 1,)),
            pltpu.SemaphoreType.DMA((N_DEV - 1,)),
            pltpu.SemaphoreType.DMA((N_DEV - 1,)),
            pltpu.SemaphoreType.DMA((N_DEV - 1,)),
        ],
        compiler_params=pltpu.CompilerParams(
            collective_id=0,
            vmem_limit_bytes=100 * 1024 * 1024,
        ),
    )(
        xf, Wq, Wk, Wv, Wo,
        jnp.asarray(_COS_T), jnp.asarray(_SIN_A), jnp.asarray(_SIN_B),
    )
    return jnp.reshape(out, (B, SQ, D))


# device time: 57494 ns/iter; 2.1579x vs baseline; 1.0416x over previous
import numpy as np
import jax
import jax.numpy as jnp
from jax import lax
from jax.experimental import pallas as pl
from jax.experimental.pallas import tpu as pltpu

N_DEV = 4
B, SQ, D = 2, 512, 1024
HQ_LOC, DH = 8, 128
ROWS = B * SQ
CHUNK = ROWS // N_DEV
SCALE = 0.08838834764831843


def _rope_tables():
    inv = 1.0 / (10000.0 ** (np.arange(0, DH, 2) / DH))
    pos = np.arange(SQ)[:, None] * inv[None, :]
    cos = np.repeat(np.cos(pos), 2, axis=-1)
    sin = np.repeat(np.sin(pos), 2, axis=-1)
    cos_t = np.tile(cos, (B, HQ_LOC))
    sin_t = np.tile(sin, (B, HQ_LOC))
    even = (np.arange(D) % 2 == 0)[None, :]
    sin_a = np.where(even, -sin_t, 0.0)
    sin_b = np.where(even, 0.0, sin_t)
    return (
        cos_t.astype(np.float32),
        sin_a.astype(np.float32),
        sin_b.astype(np.float32),
    )


_COS_T, _SIN_A, _SIN_B = _rope_tables()

_COMM = True
_BARRIER = True


def _body(
    xf_ref, wq_ref, wk_ref, wv_ref, wo_ref, cos_ref, sa_ref, sb_ref,
    out_ref,
    k_scr, v_scr, part16_scr, ag16_scr, rs_buf,
    rs_send, rs_recv, ag_send, ag_recv,
):
    my = lax.axis_index("i")

    if _COMM or _BARRIER:
        barrier_sem = pltpu.get_barrier_semaphore()
        for d in range(1, N_DEV):
            pl.semaphore_signal(
                barrier_sem, inc=1,
                device_id=((my + d) % N_DEV,),
                device_id_type=pl.DeviceIdType.MESH,
            )

    def rope(t, row_ds):
        return (
            t * cos_ref[row_ds, :]
            + pltpu.roll(t, D - 1, axis=1) * sa_ref[row_ds, :]
            + pltpu.roll(t, 1, axis=1) * sb_ref[row_ds, :]
        )

    bf16 = jnp.bfloat16
    xf16 = xf_ref[...].astype(bf16)
    wq16 = wq_ref[...].astype(bf16)
    wo16 = wo_ref[...].astype(bf16)
    all_rows = pl.ds(0, ROWS)
    k_scr[...] = rope(
        jnp.dot(
            xf16, wk_ref[...].astype(bf16),
            preferred_element_type=jnp.float32,
        ),
        all_rows,
    ).astype(bf16)
    v_scr[...] = jnp.dot(
        xf16, wv_ref[...].astype(bf16), preferred_element_type=jnp.float32
    ).astype(bf16)

    def compute_chunk(c):
        rows = pl.ds(c * CHUNK, CHUNK)
        b = c // (SQ // CHUNK)
        brows = pl.ds(b * SQ, SQ)
        q = rope(
            jnp.dot(
                (xf_ref[rows, :] * SCALE).astype(bf16),
                wq16,
                preferred_element_type=jnp.float32,
            ),
            rows,
        ).astype(bf16)
        ctx_blocks = []
        for h in range(HQ_LOC):
            c0 = h * DH
            qh = q[:, c0:c0 + DH]
            kh = k_scr[brows, c0:c0 + DH]
            vh = v_scr[brows, c0:c0 + DH]
            s = lax.dot_general(
                qh, kh, (((1,), (1,)), ((), ())),
                preferred_element_type=jnp.float32,
            )
            e = jnp.exp(s)
            denom = jnp.sum(e, axis=1, keepdims=True)
            ctxh = (
                jnp.dot(
                    e.astype(bf16), vh, preferred_element_type=jnp.float32
                )
                * (1.0 / denom)
            )
            ctx_blocks.append(ctxh.astype(bf16))
        ctx = jnp.concatenate(ctx_blocks, axis=1)
        return jnp.dot(ctx, wo16, preferred_element_type=jnp.float32)

    rs_rdmas = []
    for d in (1, 2, 3):
        c = (my + d) % N_DEV
        rows = pl.ds(c * CHUNK, CHUNK)
        part16_scr[rows, :] = compute_chunk(c).astype(jnp.bfloat16)
        if (_COMM or _BARRIER) and d == 1:
            pl.semaphore_wait(barrier_sem, N_DEV - 1)
        if _COMM:
            rdma = pltpu.make_async_remote_copy(
                src_ref=part16_scr.at[rows, :],
                dst_ref=rs_buf.at[3 - d],
                send_sem=rs_send.at[d - 1],
                recv_sem=rs_recv.at[3 - d],
                device_id=(c,),
                device_id_type=pl.DeviceIdType.MESH,
            )
            rdma.start()
            rs_rdmas.append(rdma)

    def store_out(c, h, value):
        b = c // (SQ // CHUNK)
        seq0 = (c % (SQ // CHUNK)) * CHUNK + h * HALF
        out_ref[b, pl.ds(seq0, HALF), :] = value

    HALF = CHUNK // 2
    own_rows = pl.ds(my * CHUNK, CHUNK)
    own = compute_chunk(my)
    if not _COMM:
        store_out(my, 0, own[:HALF, :])
        store_out(my, 1, own[HALF:, :])
        return
    for slot in range(N_DEV - 1):
        recv = pltpu.make_async_remote_copy(
            src_ref=part16_scr.at[pl.ds(0, CHUNK), :],
            dst_ref=rs_buf.at[slot],
            send_sem=rs_send.at[slot],
            recv_sem=rs_recv.at[slot],
            device_id=(my,),
            device_id_type=pl.DeviceIdType.MESH,
        )
        recv.wait_recv()
    ag_rdmas = []
    reduced_halves = []
    for h in (0, 1):
        hrows = pl.ds(my * CHUNK + h * HALF, HALF)
        sl = slice(h * HALF, (h + 1) * HALF)
        red = (
            own[sl, :]
            + rs_buf[0, sl, :].astype(jnp.float32)
            + rs_buf[1, sl, :].astype(jnp.float32)
            + rs_buf[2, sl, :].astype(jnp.float32)
        )
        reduced_halves.append(red)
        ag16_scr[hrows, :] = red.astype(jnp.bfloat16)
        for d in (1, 2, 3):
            peer = (my + d) % N_DEV
            rdma = pltpu.make_async_remote_copy(
                src_ref=ag16_scr.at[hrows, :],
                dst_ref=ag16_scr.at[hrows, :],
                send_sem=ag_send.at[(d - 1) * 2 + h],
                recv_sem=ag_recv.at[(3 - d) * 2 + h],
                device_id=(peer,),
                device_id_type=pl.DeviceIdType.MESH,
            )
            rdma.start()
            ag_rdmas.append(rdma)

    store_out(my, 0, reduced_halves[0])
    store_out(my, 1, reduced_halves[1])

    for e in (1, 2, 3):
        for h in (0, 1):
            recv = pltpu.make_async_remote_copy(
                src_ref=ag16_scr.at[pl.ds(0, HALF), :],
                dst_ref=ag16_scr.at[pl.ds(0, HALF), :],
                send_sem=ag_send.at[0],
                recv_sem=ag_recv.at[(e - 1) * 2 + h],
                device_id=(my,),
                device_id_type=pl.DeviceIdType.MESH,
            )
            recv.wait_recv()
            c = (my + e) % N_DEV
            prows = pl.ds(c * CHUNK + h * HALF, HALF)
            store_out(c, h, ag16_scr[prows, :].astype(jnp.float32))

    for rdma in rs_rdmas + ag_rdmas:
        rdma.wait_send()


def kernel(x, Wq, Wk, Wv, Wo):
    xf = jnp.reshape(x, (ROWS, D))
    out = pl.pallas_call(
        _body,
        out_shape=jax.ShapeDtypeStruct((B, SQ, D), jnp.float32),
        in_specs=[pl.BlockSpec(memory_space=pltpu.VMEM)] * 8,
        out_specs=pl.BlockSpec(memory_space=pltpu.VMEM),
        scratch_shapes=[
            pltpu.VMEM((ROWS, D), jnp.bfloat16),
            pltpu.VMEM((ROWS, D), jnp.bfloat16),
            pltpu.VMEM((ROWS, D), jnp.bfloat16),
            pltpu.VMEM((ROWS, D), jnp.bfloat16),
            pltpu.VMEM((N_DEV - 1, CHUNK, D), jnp.bfloat16),
            pltpu.SemaphoreType.DMA((N_DEV - 1,)),
            pltpu.SemaphoreType.DMA((N_DEV - 1,)),
            pltpu.SemaphoreType.DMA((2 * (N_DEV - 1),)),
            pltpu.SemaphoreType.DMA((2 * (N_DEV - 1),)),
        ],
        compiler_params=pltpu.CompilerParams(
            collective_id=0 if (_COMM or _BARRIER) else None,
            vmem_limit_bytes=100 * 1024 * 1024,
        ),
    )(
        xf, Wq, Wk, Wv, Wo,
        jnp.asarray(_COS_T), jnp.asarray(_SIN_A), jnp.asarray(_SIN_B),
    )
    return out
